# Initial kernel scaffold; baseline (speedup 1.0000x reference)
import functools

import jax
import jax.numpy as jnp
from jax import lax
from jax.experimental import pallas as pl
from jax.experimental.pallas import tpu as pltpu
from jax.experimental.pallas import tpu_sc as plsc

K = 100
N_ROWS = 16384
NC = 2
NS = 16
L = 16
NW = NC * NS
ROWS_PER_TILE = N_ROWS // NW
GROUPS = ROWS_PER_TILE // L
PRIOR_LAST = 1.0 - 1.0 / K


def _sigmoid(x):
    return 1.0 / (1.0 + jnp.exp(-x))


_MESH = plsc.VectorSubcoreMesh(core_axis_name="c", subcore_axis_name="s")


@functools.partial(
    pl.kernel,
    out_type=jax.ShapeDtypeStruct((NW, L), jnp.float32),
    mesh=_MESH,
    scratch_types=[
        pltpu.VMEM((ROWS_PER_TILE, K), jnp.float32),
        pltpu.VMEM((ROWS_PER_TILE,), jnp.int32),
        pltpu.VMEM((L,), jnp.float32),
    ],
)
def _sc_partials(pred_hbm, y_hbm, out_hbm, block_v, y_v, pv_v):
    wid = lax.axis_index("s") * NC + lax.axis_index("c")
    base = wid * ROWS_PER_TILE
    pltpu.sync_copy(pred_hbm.at[pl.ds(base, ROWS_PER_TILE), :], block_v)
    pltpu.sync_copy(y_hbm.at[pl.ds(base, ROWS_PER_TILE)], y_v)

    iota = lax.iota(jnp.int32, L)
    zero = jnp.zeros((L,), jnp.float32)

    def group_body(g, carry):
        acc_a, acc_s, acc_n = carry
        rows = g * L + iota
        yv = y_v[pl.ds(g * L, L)]
        diag = plsc.load_gather(block_v, [rows, yv])
        last = plsc.load_gather(block_v, [rows, jnp.full((L,), K - 1, jnp.int32)])
        m99 = yv == (K - 1)

        sa = _sigmoid(-diag) + _sigmoid(last)
        acc_a = acc_a + jnp.where(m99, 0.0, sa)
        acc_n = acc_n + jnp.where(m99, 1.0, 0.0)

        def col_body(j, rm):
            col = plsc.load_gather(block_v, [rows, jnp.full((L,), j, jnp.int32)])
            return jnp.maximum(rm, col)

        rmax = lax.fori_loop(0, K - 1, col_body,
                             jnp.full((L,), -jnp.inf, jnp.float32))
        sv = _sigmoid(rmax) + _sigmoid(-last)
        acc_s = acc_s + jnp.where(m99, sv, 0.0)
        return acc_a, acc_s, acc_n

    acc_a, acc_s, acc_n = lax.fori_loop(
        0, GROUPS, group_body, (zero, zero, zero))

    p_a = jnp.sum(acc_a)
    p_s = jnp.sum(acc_s)
    p_n = jnp.sum(acc_n)
    pv = jnp.where(iota == 0, p_a,
                   jnp.where(iota == 1, p_s,
                             jnp.where(iota == 2, p_n, 0.0)))
    pv_v[...] = pv
    pltpu.sync_copy(pv_v, out_hbm.at[wid])


def _combine_body(p_ref, o_ref):
    x = p_ref[...]
    lane = lax.broadcasted_iota(jnp.int32, x.shape, 1)
    sum_a = jnp.sum(jnp.where(lane == 0, x, 0.0))
    s99 = jnp.sum(jnp.where(lane == 1, x, 0.0))
    n99 = jnp.sum(jnp.where(lane == 2, x, 0.0))
    mean_v = s99 / jnp.maximum(n99, 1.0)
    core = jnp.abs(mean_v - PRIOR_LAST)
    o_ref[0, 0] = sum_a / N_ROWS + (n99 / N_ROWS) * core


_combine = pl.pallas_call(
    _combine_body,
    out_shape=jax.ShapeDtypeStruct((1, 1), jnp.float32),
    out_specs=pl.BlockSpec(memory_space=pltpu.SMEM),
)


def kernel(pred, y):
    partials = _sc_partials(pred, y)
    return _combine(partials)[0, 0]


# speedup vs baseline: 3.4493x; 3.4493x over previous
import functools

import jax
import jax.numpy as jnp
from jax import lax
from jax.experimental import pallas as pl
from jax.experimental.pallas import tpu as pltpu
from jax.experimental.pallas import tpu_sc as plsc

K = 100
N_ROWS = 16384
NC = 2
NS = 16
L = 16
NW = NC * NS
ROWS_PER_TILE = N_ROWS // NW
GROUPS = ROWS_PER_TILE // L
PRIOR_LAST = 1.0 - 1.0 / K


def _sigmoid(x):
    return 1.0 / (1.0 + jnp.exp(-x))


_MESH = plsc.VectorSubcoreMesh(core_axis_name="c", subcore_axis_name="s")


@functools.partial(
    pl.kernel,
    out_type=jax.ShapeDtypeStruct((NW, L), jnp.float32),
    mesh=_MESH,
    compiler_params=pltpu.CompilerParams(needs_layout_passes=False),
    scratch_types=[
        pltpu.VMEM((ROWS_PER_TILE * K,), jnp.float32),
        pltpu.VMEM((ROWS_PER_TILE,), jnp.int32),
        pltpu.VMEM((L,), jnp.float32),
    ],
)
def _sc_partials(pred_hbm, y_hbm, out_hbm, block_v, y_v, pv_v):
    wid = lax.axis_index("s") * NC + lax.axis_index("c")
    base = wid * ROWS_PER_TILE
    pltpu.sync_copy(pred_hbm.at[pl.ds(base * K, ROWS_PER_TILE * K)], block_v)
    pltpu.sync_copy(y_hbm.at[pl.ds(base, ROWS_PER_TILE)], y_v)

    iota = lax.iota(jnp.int32, L)
    zero = jnp.zeros((L,), jnp.float32)

    def group_body(g, carry):
        acc_a, acc_s, acc_n = carry
        row_base = (g * L + iota) * K
        yv = y_v[pl.ds(g * L, L)]
        diag = plsc.load_gather(block_v, [row_base + yv])
        last = plsc.load_gather(block_v, [row_base + (K - 1)])
        m99 = yv == (K - 1)

        sa = _sigmoid(-diag) + _sigmoid(last)
        acc_a = acc_a + jnp.where(m99, 0.0, sa)
        acc_n = acc_n + jnp.where(m99, 1.0, 0.0)

        def col_body(j, rm):
            col = plsc.load_gather(block_v, [row_base + j])
            return jnp.maximum(rm, col)

        rmax = lax.fori_loop(0, K - 1, col_body,
                             jnp.full((L,), -jnp.inf, jnp.float32))
        sv = _sigmoid(rmax) + _sigmoid(-last)
        acc_s = acc_s + jnp.where(m99, sv, 0.0)
        return acc_a, acc_s, acc_n

    acc_a, acc_s, acc_n = lax.fori_loop(
        0, GROUPS, group_body, (zero, zero, zero))

    p_a = jnp.sum(acc_a)
    p_s = jnp.sum(acc_s)
    p_n = jnp.sum(acc_n)
    pv = jnp.where(iota == 0, p_a,
                   jnp.where(iota == 1, p_s,
                             jnp.where(iota == 2, p_n, 0.0)))
    pv_v[...] = pv
    pltpu.sync_copy(pv_v, out_hbm.at[wid])


def _combine_body(p_ref, o_ref):
    x = p_ref[...]
    lane = lax.broadcasted_iota(jnp.int32, x.shape, 1)
    sum_a = jnp.sum(jnp.where(lane == 0, x, 0.0))
    s99 = jnp.sum(jnp.where(lane == 1, x, 0.0))
    n99 = jnp.sum(jnp.where(lane == 2, x, 0.0))
    mean_v = s99 / jnp.maximum(n99, 1.0)
    core = jnp.abs(mean_v - PRIOR_LAST)
    o_ref[0, 0] = sum_a / N_ROWS + (n99 / N_ROWS) * core


_combine = pl.pallas_call(
    _combine_body,
    out_shape=jax.ShapeDtypeStruct((1, 1), jnp.float32),
    out_specs=pl.BlockSpec(memory_space=pltpu.SMEM),
)


def kernel(pred, y):
    partials = _sc_partials(pred.reshape(-1), y)
    return _combine(partials)[0, 0]


# speedup vs baseline: 3.9199x; 1.1365x over previous
import functools

import jax
import jax.numpy as jnp
from jax import lax
from jax.experimental import pallas as pl
from jax.experimental.pallas import tpu as pltpu
from jax.experimental.pallas import tpu_sc as plsc

K = 100
N_ROWS = 16384
NC = 2
NS = 16
L = 16
NW = NC * NS
ROWS_PER_TILE = N_ROWS // NW
GROUPS = ROWS_PER_TILE // L
PRIOR_LAST = 1.0 - 1.0 / K


def _sigmoid(x):
    return 1.0 / (1.0 + jnp.exp(-x))


_MESH = plsc.VectorSubcoreMesh(core_axis_name="c", subcore_axis_name="s")


@functools.partial(
    pl.kernel,
    out_type=jax.ShapeDtypeStruct((NW, L), jnp.float32),
    mesh=_MESH,
    compiler_params=pltpu.CompilerParams(needs_layout_passes=False),
    scratch_types=[
        pltpu.VMEM((ROWS_PER_TILE * K,), jnp.float32),
        pltpu.VMEM((ROWS_PER_TILE,), jnp.int32),
        pltpu.VMEM((L,), jnp.float32),
    ],
)
def _sc_partials(pred_hbm, y_hbm, out_hbm, block_v, y_v, pv_v):
    wid = lax.axis_index("s") * NC + lax.axis_index("c")
    base = wid * ROWS_PER_TILE
    pltpu.sync_copy(pred_hbm.at[pl.ds(base * K, ROWS_PER_TILE * K)], block_v)
    pltpu.sync_copy(y_hbm.at[pl.ds(base, ROWS_PER_TILE)], y_v)

    iota = lax.iota(jnp.int32, L)
    zero = jnp.zeros((L,), jnp.float32)

    def group_body(g, carry):
        acc_a, acc_s, acc_n = carry
        row_base = (g * L + iota) * K
        yv = y_v[pl.ds(g * L, L)]
        diag = plsc.load_gather(block_v, [row_base + yv])
        last = plsc.load_gather(block_v, [row_base + (K - 1)])
        m99 = yv == (K - 1)

        sa = _sigmoid(-diag) + _sigmoid(last)
        acc_a = acc_a + jnp.where(m99, 0.0, sa)
        acc_n = acc_n + jnp.where(m99, 1.0, 0.0)

        def with_max(acc):
            def col_body(j, rm):
                col = plsc.load_gather(block_v, [row_base + j])
                return jnp.maximum(rm, col)

            rmax = lax.fori_loop(1, K - 1, col_body,
                                 plsc.load_gather(block_v, [row_base]))
            sv = _sigmoid(rmax) + _sigmoid(-last)
            return acc + jnp.where(m99, sv, 0.0)

        acc_s = lax.cond(jnp.any(m99), with_max, lambda acc: acc, acc_s)
        return acc_a, acc_s, acc_n

    acc_a, acc_s, acc_n = lax.fori_loop(
        0, GROUPS, group_body, (zero, zero, zero))

    p_a = jnp.sum(acc_a)
    p_s = jnp.sum(acc_s)
    p_n = jnp.sum(acc_n)
    pv = jnp.where(iota == 0, p_a,
                   jnp.where(iota == 1, p_s,
                             jnp.where(iota == 2, p_n, 0.0)))
    pv_v[...] = pv
    pltpu.sync_copy(pv_v, out_hbm.at[wid])


def _combine_body(p_ref, o_ref):
    x = p_ref[...]
    lane = lax.broadcasted_iota(jnp.int32, x.shape, 1)
    sum_a = jnp.sum(jnp.where(lane == 0, x, 0.0))
    s99 = jnp.sum(jnp.where(lane == 1, x, 0.0))
    n99 = jnp.sum(jnp.where(lane == 2, x, 0.0))
    mean_v = s99 / jnp.maximum(n99, 1.0)
    core = jnp.abs(mean_v - PRIOR_LAST)
    o_ref[0, 0] = sum_a / N_ROWS + (n99 / N_ROWS) * core


_combine = pl.pallas_call(
    _combine_body,
    out_shape=jax.ShapeDtypeStruct((1, 1), jnp.float32),
    out_specs=pl.BlockSpec(memory_space=pltpu.SMEM),
)


def kernel(pred, y):
    partials = _sc_partials(pred.reshape(-1), y)
    return _combine(partials)[0, 0]


# speedup vs baseline: 4.5564x; 1.1624x over previous
import functools

import jax
import jax.numpy as jnp
from jax import lax
from jax.experimental import pallas as pl
from jax.experimental.pallas import tpu as pltpu
from jax.experimental.pallas import tpu_sc as plsc

K = 100
N_ROWS = 16384
NC = 2
NS = 16
L = 16
NW = NC * NS
ROWS_PER_TILE = N_ROWS // NW
GROUPS = ROWS_PER_TILE // L
PRIOR_LAST = 1.0 - 1.0 / K


def _sigmoid(x):
    return 1.0 / (1.0 + jnp.exp(-x))


_MESH = plsc.VectorSubcoreMesh(core_axis_name="c", subcore_axis_name="s")


@functools.partial(
    pl.kernel,
    out_type=jax.ShapeDtypeStruct((NW, L), jnp.float32),
    mesh=_MESH,
    compiler_params=pltpu.CompilerParams(needs_layout_passes=False),
    scratch_types=[
        pltpu.VMEM((ROWS_PER_TILE, K), jnp.float32),
        pltpu.VMEM((ROWS_PER_TILE,), jnp.int32),
        pltpu.VMEM((L,), jnp.float32),
    ],
)
def _sc_partials(pred_hbm, y_hbm, out_hbm, block_v, y_v, pv_v):
    wid = lax.axis_index("s") * NC + lax.axis_index("c")
    base = wid * ROWS_PER_TILE
    pltpu.sync_copy(pred_hbm.at[pl.ds(base, ROWS_PER_TILE), :], block_v)
    pltpu.sync_copy(y_hbm.at[pl.ds(base, ROWS_PER_TILE)], y_v)

    iota = lax.iota(jnp.int32, L)
    zero = jnp.zeros((L,), jnp.float32)

    def group_body(g, carry):
        acc_a, acc_s, acc_n = carry
        rows = g * L + iota
        yv = y_v[pl.ds(g * L, L)]
        diag = plsc.load_gather(block_v, [rows, yv])
        last = plsc.load_gather(block_v, [rows, jnp.full((L,), K - 1, jnp.int32)])
        m99 = yv == (K - 1)

        sa = _sigmoid(-diag) + _sigmoid(last)
        acc_a = acc_a + jnp.where(m99, 0.0, sa)
        acc_n = acc_n + jnp.where(m99, 1.0, 0.0)

        def with_max(acc):
            def col_body(j, rm):
                col = plsc.load_gather(block_v, [rows, jnp.full((L,), j, jnp.int32)])
                return jnp.maximum(rm, col)

            rmax = lax.fori_loop(1, K - 1, col_body,
                                 plsc.load_gather(block_v, [rows, jnp.zeros((L,), jnp.int32)]))
            sv = _sigmoid(rmax) + _sigmoid(-last)
            return acc + jnp.where(m99, sv, 0.0)

        acc_s = lax.cond(jnp.any(m99), with_max, lambda acc: acc, acc_s)
        return acc_a, acc_s, acc_n

    acc_a, acc_s, acc_n = lax.fori_loop(
        0, GROUPS, group_body, (zero, zero, zero))

    p_a = jnp.sum(acc_a)
    p_s = jnp.sum(acc_s)
    p_n = jnp.sum(acc_n)
    pv = jnp.where(iota == 0, p_a,
                   jnp.where(iota == 1, p_s,
                             jnp.where(iota == 2, p_n, 0.0)))
    pv_v[...] = pv
    pltpu.sync_copy(pv_v, out_hbm.at[wid])


def _combine_body(p_ref, o_ref):
    x = p_ref[...]
    lane = lax.broadcasted_iota(jnp.int32, x.shape, 1)
    sum_a = jnp.sum(jnp.where(lane == 0, x, 0.0))
    s99 = jnp.sum(jnp.where(lane == 1, x, 0.0))
    n99 = jnp.sum(jnp.where(lane == 2, x, 0.0))
    mean_v = s99 / jnp.maximum(n99, 1.0)
    core = jnp.abs(mean_v - PRIOR_LAST)
    o_ref[0, 0] = sum_a / N_ROWS + (n99 / N_ROWS) * core


_combine = pl.pallas_call(
    _combine_body,
    out_shape=jax.ShapeDtypeStruct((1, 1), jnp.float32),
    out_specs=pl.BlockSpec(memory_space=pltpu.SMEM),
)


def kernel(pred, y):
    partials = _sc_partials(pred, y)
    return _combine(partials)[0, 0]


# speedup vs baseline: 4.5578x; 1.0003x over previous
import functools

import jax
import jax.numpy as jnp
from jax import lax
from jax.experimental import pallas as pl
from jax.experimental.pallas import tpu as pltpu
from jax.experimental.pallas import tpu_sc as plsc

K = 100
N_ROWS = 16384
NC = 2
NS = 16
L = 16
NW = NC * NS
ROWS_PER_TILE = N_ROWS // NW
GROUPS = ROWS_PER_TILE // L
PRIOR_LAST = 1.0 - 1.0 / K


def _sigmoid(x):
    return 1.0 / (1.0 + jnp.exp(-x))


_MESH = plsc.VectorSubcoreMesh(core_axis_name="c", subcore_axis_name="s")


@functools.partial(
    pl.kernel,
    out_type=jax.ShapeDtypeStruct((NW, L), jnp.float32),
    mesh=_MESH,
    compiler_params=pltpu.CompilerParams(needs_layout_passes=False,
                                         use_tc_tiling_on_sc=True),
    scratch_types=[
        pltpu.VMEM((ROWS_PER_TILE, K), jnp.float32),
        pltpu.VMEM((ROWS_PER_TILE,), jnp.int32),
        pltpu.VMEM((L,), jnp.float32),
    ],
)
def _sc_partials(pred_hbm, y_hbm, out_hbm, block_v, y_v, pv_v):
    wid = lax.axis_index("s") * NC + lax.axis_index("c")
    base = wid * ROWS_PER_TILE
    pltpu.sync_copy(pred_hbm.at[pl.ds(base, ROWS_PER_TILE), :], block_v)
    pltpu.sync_copy(y_hbm.at[pl.ds(base, ROWS_PER_TILE)], y_v)

    iota = lax.iota(jnp.int32, L)
    zero = jnp.zeros((L,), jnp.float32)

    def group_body(g, carry):
        acc_a, acc_s, acc_n = carry
        rows = g * L + iota
        yv = y_v[pl.ds(g * L, L)]
        diag = plsc.load_gather(block_v, [rows, yv])
        last = plsc.load_gather(block_v, [rows, jnp.full((L,), K - 1, jnp.int32)])
        m99 = yv == (K - 1)

        sa = _sigmoid(-diag) + _sigmoid(last)
        acc_a = acc_a + jnp.where(m99, 0.0, sa)
        acc_n = acc_n + jnp.where(m99, 1.0, 0.0)

        def with_max(acc):
            def col_body(j, rm):
                col = plsc.load_gather(block_v, [rows, jnp.full((L,), j, jnp.int32)])
                return jnp.maximum(rm, col)

            rmax = lax.fori_loop(1, K - 1, col_body,
                                 plsc.load_gather(block_v, [rows, jnp.zeros((L,), jnp.int32)]))
            sv = _sigmoid(rmax) + _sigmoid(-last)
            return acc + jnp.where(m99, sv, 0.0)

        acc_s = lax.cond(jnp.any(m99), with_max, lambda acc: acc, acc_s)
        return acc_a, acc_s, acc_n

    acc_a, acc_s, acc_n = lax.fori_loop(
        0, GROUPS, group_body, (zero, zero, zero))

    p_a = jnp.sum(acc_a)
    p_s = jnp.sum(acc_s)
    p_n = jnp.sum(acc_n)
    pv = jnp.where(iota == 0, p_a,
                   jnp.where(iota == 1, p_s,
                             jnp.where(iota == 2, p_n, 0.0)))
    pv_v[...] = pv
    pltpu.sync_copy(pv_v, out_hbm.at[wid])


def _combine_body(p_ref, o_ref):
    x = p_ref[...]
    lane = lax.broadcasted_iota(jnp.int32, x.shape, 1)
    sum_a = jnp.sum(jnp.where(lane == 0, x, 0.0))
    s99 = jnp.sum(jnp.where(lane == 1, x, 0.0))
    n99 = jnp.sum(jnp.where(lane == 2, x, 0.0))
    mean_v = s99 / jnp.maximum(n99, 1.0)
    core = jnp.abs(mean_v - PRIOR_LAST)
    o_ref[0, 0] = sum_a / N_ROWS + (n99 / N_ROWS) * core


_combine = pl.pallas_call(
    _combine_body,
    out_shape=jax.ShapeDtypeStruct((1, 1), jnp.float32),
    out_specs=pl.BlockSpec(memory_space=pltpu.SMEM),
)


def kernel(pred, y):
    partials = _sc_partials(pred, y)
    return _combine(partials)[0, 0]


# speedup vs baseline: 6.9533x; 1.5256x over previous
import functools

import jax
import jax.numpy as jnp
from jax import lax
from jax.experimental import pallas as pl
from jax.experimental.pallas import tpu as pltpu
from jax.experimental.pallas import tpu_sc as plsc

K = 100
N_ROWS = 16384
NC = 2
NS = 16
L = 16
NW = NC * NS
ROWS_PER_TILE = N_ROWS // NW
GROUPS = ROWS_PER_TILE // L
PRIOR_LAST = 1.0 - 1.0 / K


def _sigmoid(x):
    return 1.0 / (1.0 + jnp.exp(-x))


_MESH = plsc.VectorSubcoreMesh(core_axis_name="c", subcore_axis_name="s")


@functools.partial(
    pl.kernel,
    out_type=jax.ShapeDtypeStruct((NW, L), jnp.float32),
    mesh=_MESH,
    compiler_params=pltpu.CompilerParams(needs_layout_passes=False,
                                         use_tc_tiling_on_sc=True),
    scratch_types=[
        pltpu.VMEM((K, ROWS_PER_TILE), jnp.float32),
        pltpu.VMEM((ROWS_PER_TILE,), jnp.int32),
        pltpu.VMEM((L,), jnp.float32),
    ],
)
def _sc_partials(predt_hbm, y_hbm, out_hbm, block_v, y_v, pv_v):
    wid = lax.axis_index("s") * NC + lax.axis_index("c")
    base = wid * ROWS_PER_TILE
    pltpu.sync_copy(predt_hbm.at[:, pl.ds(base, ROWS_PER_TILE)], block_v)
    pltpu.sync_copy(y_hbm.at[pl.ds(base, ROWS_PER_TILE)], y_v)

    iota = lax.iota(jnp.int32, L)
    zero = jnp.zeros((L,), jnp.float32)

    def group_body(g, carry):
        acc_a, acc_s, acc_n = carry
        off = g * L
        rloc = off + iota
        yv = y_v[pl.ds(off, L)]
        diag = plsc.load_gather(block_v, [yv, rloc])
        last = block_v[K - 1, pl.ds(off, L)]
        m99 = yv == (K - 1)

        sa = _sigmoid(-diag) + _sigmoid(last)
        acc_a = acc_a + jnp.where(m99, 0.0, sa)
        acc_n = acc_n + jnp.where(m99, 1.0, 0.0)

        def with_max(acc):
            def col_body(j, rm):
                return jnp.maximum(rm, block_v[j, pl.ds(off, L)])

            rmax = lax.fori_loop(1, K - 1, col_body, block_v[0, pl.ds(off, L)])
            sv = _sigmoid(rmax) + _sigmoid(-last)
            return acc + jnp.where(m99, sv, 0.0)

        acc_s = lax.cond(jnp.any(m99), with_max, lambda acc: acc, acc_s)
        return acc_a, acc_s, acc_n

    acc_a, acc_s, acc_n = lax.fori_loop(
        0, GROUPS, group_body, (zero, zero, zero))

    p_a = jnp.sum(acc_a)
    p_s = jnp.sum(acc_s)
    p_n = jnp.sum(acc_n)
    pv = jnp.where(iota == 0, p_a,
                   jnp.where(iota == 1, p_s,
                             jnp.where(iota == 2, p_n, 0.0)))
    pv_v[...] = pv
    pltpu.sync_copy(pv_v, out_hbm.at[wid])


def _combine_body(p_ref, o_ref):
    x = p_ref[...]
    lane = lax.broadcasted_iota(jnp.int32, x.shape, 1)
    sum_a = jnp.sum(jnp.where(lane == 0, x, 0.0))
    s99 = jnp.sum(jnp.where(lane == 1, x, 0.0))
    n99 = jnp.sum(jnp.where(lane == 2, x, 0.0))
    mean_v = s99 / jnp.maximum(n99, 1.0)
    core = jnp.abs(mean_v - PRIOR_LAST)
    o_ref[0, 0] = sum_a / N_ROWS + (n99 / N_ROWS) * core


_combine = pl.pallas_call(
    _combine_body,
    out_shape=jax.ShapeDtypeStruct((1, 1), jnp.float32),
    out_specs=pl.BlockSpec(memory_space=pltpu.SMEM),
)


def kernel(pred, y):
    partials = _sc_partials(pred.T, y)
    return _combine(partials)[0, 0]


# speedup vs baseline: 7.9316x; 1.1407x over previous
import functools

import jax
import jax.numpy as jnp
from jax import lax
from jax.experimental import pallas as pl
from jax.experimental.pallas import tpu as pltpu
from jax.experimental.pallas import tpu_sc as plsc

K = 100
N_ROWS = 16384
NC = 2
NS = 16
L = 16
NW = NC * NS
ROWS_PER_TILE = N_ROWS // NW
GROUPS = ROWS_PER_TILE // L
PRIOR_LAST = 1.0 - 1.0 / K


def _sigmoid(x):
    return 1.0 / (1.0 + jnp.exp(-x))


_MESH = plsc.VectorSubcoreMesh(core_axis_name="c", subcore_axis_name="s")


@functools.partial(
    pl.kernel,
    out_type=jax.ShapeDtypeStruct((NW, L), jnp.float32),
    mesh=_MESH,
    compiler_params=pltpu.CompilerParams(needs_layout_passes=False,
                                         use_tc_tiling_on_sc=True),
    scratch_types=[
        pltpu.VMEM((K, ROWS_PER_TILE // 2), jnp.float32),
        pltpu.VMEM((K, ROWS_PER_TILE // 2), jnp.float32),
        pltpu.VMEM((ROWS_PER_TILE,), jnp.int32),
        pltpu.VMEM((L,), jnp.float32),
        pltpu.SemaphoreType.DMA,
        pltpu.SemaphoreType.DMA,
    ],
)
def _sc_partials(predt_hbm, y_hbm, out_hbm, buf0, buf1, y_v, pv_v, sem0, sem1):
    wid = lax.axis_index("s") * NC + lax.axis_index("c")
    base = wid * ROWS_PER_TILE
    half = ROWS_PER_TILE // 2
    bufs, sems = (buf0, buf1), (sem0, sem1)
    copies = [
        pltpu.async_copy(predt_hbm.at[:, pl.ds(base + h * half, half)],
                         bufs[h], sems[h])
        for h in range(2)
    ]
    pltpu.sync_copy(y_hbm.at[pl.ds(base, ROWS_PER_TILE)], y_v)

    iota = lax.iota(jnp.int32, L)
    zero = jnp.zeros((L,), jnp.float32)

    def make_group_body(block_v, y_off):
        def group_body(g, carry):
            acc_a, acc_s, acc_n = carry
            off = g * L
            rloc = off + iota
            yv = y_v[pl.ds(y_off + off, L)]
            diag = plsc.load_gather(block_v, [yv, rloc])
            last = block_v[K - 1, pl.ds(off, L)]
            m99 = yv == (K - 1)

            sa = _sigmoid(-diag) + _sigmoid(last)
            acc_a = acc_a + jnp.where(m99, 0.0, sa)
            acc_n = acc_n + jnp.where(m99, 1.0, 0.0)

            def with_max(acc):
                def col_body(t, rm):
                    j = 7 * t
                    for d in range(1, 8):
                        rm = jnp.maximum(rm, block_v[j + d, pl.ds(off, L)])
                    return rm

                rmax = lax.fori_loop(0, 14, col_body, block_v[0, pl.ds(off, L)])
                sv = _sigmoid(rmax) + _sigmoid(-last)
                return acc + jnp.where(m99, sv, 0.0)

            acc_s = lax.cond(jnp.any(m99), with_max, lambda acc: acc, acc_s)
            return acc_a, acc_s, acc_n

        return group_body

    acc = (zero, zero, zero)
    for h in range(2):
        copies[h].wait()
        acc = lax.fori_loop(0, half // L,
                            make_group_body(bufs[h], h * half), acc)
    acc_a, acc_s, acc_n = acc

    p_a = jnp.sum(acc_a)
    p_s = jnp.sum(acc_s)
    p_n = jnp.sum(acc_n)
    pv = jnp.where(iota == 0, p_a,
                   jnp.where(iota == 1, p_s,
                             jnp.where(iota == 2, p_n, 0.0)))
    pv_v[...] = pv
    pltpu.sync_copy(pv_v, out_hbm.at[wid])


def _combine_body(p_ref, o_ref):
    x = p_ref[...]
    lane = lax.broadcasted_iota(jnp.int32, x.shape, 1)
    sum_a = jnp.sum(jnp.where(lane == 0, x, 0.0))
    s99 = jnp.sum(jnp.where(lane == 1, x, 0.0))
    n99 = jnp.sum(jnp.where(lane == 2, x, 0.0))
    mean_v = s99 / jnp.maximum(n99, 1.0)
    core = jnp.abs(mean_v - PRIOR_LAST)
    o_ref[0, 0] = sum_a / N_ROWS + (n99 / N_ROWS) * core


_combine = pl.pallas_call(
    _combine_body,
    out_shape=jax.ShapeDtypeStruct((1, 1), jnp.float32),
    out_specs=pl.BlockSpec(memory_space=pltpu.SMEM),
)


def kernel(pred, y):
    partials = _sc_partials(pred.T, y)
    return _combine(partials)[0, 0]


# speedup vs baseline: 7.9676x; 1.0045x over previous
import functools

import jax
import jax.numpy as jnp
from jax import lax
from jax.experimental import pallas as pl
from jax.experimental.pallas import tpu as pltpu
from jax.experimental.pallas import tpu_sc as plsc

K = 100
N_ROWS = 16384
NC = 2
NS = 16
L = 16
NW = NC * NS
ROWS_PER_TILE = N_ROWS // NW
GROUPS = ROWS_PER_TILE // L
PRIOR_LAST = 1.0 - 1.0 / K


def _sigmoid(x):
    return 1.0 / (1.0 + jnp.exp(-x))


_MESH = plsc.VectorSubcoreMesh(core_axis_name="c", subcore_axis_name="s")


@functools.partial(
    pl.kernel,
    out_type=jax.ShapeDtypeStruct((NW, L), jnp.float32),
    mesh=_MESH,
    compiler_params=pltpu.CompilerParams(needs_layout_passes=False,
                                         use_tc_tiling_on_sc=True),
    scratch_types=[
        pltpu.VMEM((K, ROWS_PER_TILE // 4), jnp.float32),
        pltpu.VMEM((K, ROWS_PER_TILE // 4), jnp.float32),
        pltpu.VMEM((K, ROWS_PER_TILE // 4), jnp.float32),
        pltpu.VMEM((K, ROWS_PER_TILE // 4), jnp.float32),
        pltpu.VMEM((ROWS_PER_TILE,), jnp.int32),
        pltpu.VMEM((L,), jnp.float32),
        pltpu.SemaphoreType.DMA,
        pltpu.SemaphoreType.DMA,
        pltpu.SemaphoreType.DMA,
        pltpu.SemaphoreType.DMA,
        pltpu.SemaphoreType.DMA,
    ],
)
def _sc_partials(predt_hbm, y_hbm, out_hbm, buf0, buf1, buf2, buf3,
                 y_v, pv_v, sem0, sem1, sem2, sem3, ysem):
    wid = lax.axis_index("s") * NC + lax.axis_index("c")
    base = wid * ROWS_PER_TILE
    half = ROWS_PER_TILE // 4
    bufs, sems = (buf0, buf1, buf2, buf3), (sem0, sem1, sem2, sem3)
    ycopy = pltpu.async_copy(y_hbm.at[pl.ds(base, ROWS_PER_TILE)], y_v, ysem)
    copies = [
        pltpu.async_copy(predt_hbm.at[:, pl.ds(base + h * half, half)],
                         bufs[h], sems[h])
        for h in range(4)
    ]
    ycopy.wait()

    iota = lax.iota(jnp.int32, L)
    zero = jnp.zeros((L,), jnp.float32)

    def make_group_body(block_v, y_off):
        def group_body(g, carry):
            acc_a, acc_s, acc_n = carry
            off = g * L
            rloc = off + iota
            yv = y_v[pl.ds(y_off + off, L)]
            diag = plsc.load_gather(block_v, [yv, rloc])
            last = block_v[K - 1, pl.ds(off, L)]
            m99 = yv == (K - 1)

            sa = _sigmoid(-diag) + _sigmoid(last)
            acc_a = acc_a + jnp.where(m99, 0.0, sa)
            acc_n = acc_n + jnp.where(m99, 1.0, 0.0)

            def with_max(acc):
                def col_body(t, rm):
                    j = 7 * t
                    for d in range(1, 8):
                        rm = jnp.maximum(rm, block_v[j + d, pl.ds(off, L)])
                    return rm

                rmax = lax.fori_loop(0, 14, col_body, block_v[0, pl.ds(off, L)])
                sv = _sigmoid(rmax) + _sigmoid(-last)
                return acc + jnp.where(m99, sv, 0.0)

            acc_s = lax.cond(jnp.any(m99), with_max, lambda acc: acc, acc_s)
            return acc_a, acc_s, acc_n

        return group_body

    acc = (zero, zero, zero)
    for h in range(4):
        copies[h].wait()
        acc = lax.fori_loop(0, half // L,
                            make_group_body(bufs[h], h * half), acc)
    acc_a, acc_s, acc_n = acc

    p_a = jnp.sum(acc_a)
    p_s = jnp.sum(acc_s)
    p_n = jnp.sum(acc_n)
    pv = jnp.where(iota == 0, p_a,
                   jnp.where(iota == 1, p_s,
                             jnp.where(iota == 2, p_n, 0.0)))
    pv_v[...] = pv
    pltpu.sync_copy(pv_v, out_hbm.at[wid])


def _combine_body(p_ref, o_ref):
    x = p_ref[...]
    lane = lax.broadcasted_iota(jnp.int32, x.shape, 1)
    sum_a = jnp.sum(jnp.where(lane == 0, x, 0.0))
    s99 = jnp.sum(jnp.where(lane == 1, x, 0.0))
    n99 = jnp.sum(jnp.where(lane == 2, x, 0.0))
    mean_v = s99 / jnp.maximum(n99, 1.0)
    core = jnp.abs(mean_v - PRIOR_LAST)
    o_ref[0, 0] = sum_a / N_ROWS + (n99 / N_ROWS) * core


_combine = pl.pallas_call(
    _combine_body,
    out_shape=jax.ShapeDtypeStruct((1, 1), jnp.float32),
    out_specs=pl.BlockSpec(memory_space=pltpu.SMEM),
)


def kernel(pred, y):
    partials = _sc_partials(pred.T, y)
    return _combine(partials)[0, 0]
